# Initial kernel scaffold; baseline (speedup 1.0000x reference)
import functools

import jax
import jax.numpy as jnp
from jax import lax
from jax.experimental import pallas as pl
from jax.experimental.pallas import tpu as pltpu
from jax.experimental.pallas import tpu_sc as plsc

N = 10000
E = 320000
D = 128

NW = 32
RPW = 320
NPAD = NW * RPW
C = 128
IPTR_BUF = 336
IPTR_LEN = (NW - 1) * RPW + IPTR_BUF
EP = E + 2 * C


def _body(x_hbm, iptr_hbm, idx_hbm, val_hbm, out_hbm,
          iptr_v, idx_v, val_v, rows_v, acc_v, sem):
    cid = lax.axis_index("c")
    sid = lax.axis_index("s")
    w = sid * 2 + cid
    r0 = pl.multiple_of(w * RPW, RPW)

    pltpu.sync_copy(iptr_hbm.at[pl.ds(r0, IPTR_BUF)], iptr_v)

    zeros16 = jnp.zeros((16,), jnp.float32)

    def zero_body(i, carry):
        acc_v[pl.ds(pl.multiple_of(i * 16, 16), 16)] = zeros16
        return carry

    lax.fori_loop(0, RPW * D // 16, zero_body, 0)

    e_lo = iptr_v[0]
    e_hi = iptr_v[RPW]
    a0 = lax.bitwise_and(e_lo, jnp.int32(-16))
    nch = (e_hi - a0 + (C - 1)) // C

    def chunk_body(k, r):
        s = pl.multiple_of(a0 + k * C, 16)
        pltpu.sync_copy(idx_hbm.at[pl.ds(s, C)], idx_v)
        pltpu.sync_copy(val_hbm.at[pl.ds(s, C)], val_v)
        pltpu.async_copy(x_hbm.at[idx_v], rows_v, sem).wait()

        p_lo = jnp.maximum(e_lo, s)
        p_hi = jnp.minimum(e_hi, s + C)

        def edge_body(p, r):
            r = lax.while_loop(lambda rr: iptr_v[rr + 1] <= p,
                               lambda rr: rr + 1, r)
            j = p - s
            vv = jnp.full((16,), val_v[j])
            ab = r * D
            for d in range(8):
                xv = rows_v[j, pl.ds(d * 16, 16)]
                plsc.addupdate(acc_v.at[pl.ds(ab + d * 16, 16)], xv * vv)
            return r

        return lax.fori_loop(p_lo, p_hi, edge_body, r)

    lax.fori_loop(0, nch, chunk_body, jnp.int32(0))

    pltpu.sync_copy(acc_v, out_hbm.at[pl.ds(pl.multiple_of(r0 * D, 16),
                                            RPW * D)])


@functools.partial(
    pl.kernel,
    out_type=jax.ShapeDtypeStruct((NPAD * D,), jnp.float32),
    mesh=plsc.VectorSubcoreMesh(core_axis_name="c", subcore_axis_name="s"),
    scratch_types=[
        pltpu.VMEM((IPTR_BUF,), jnp.int32),
        pltpu.VMEM((C,), jnp.int32),
        pltpu.VMEM((C,), jnp.float32),
        pltpu.VMEM((C, D), jnp.float32),
        pltpu.VMEM((RPW * D,), jnp.float32),
        pltpu.SemaphoreType.DMA,
    ],
)
def _sc_spmm(x_hbm, iptr_hbm, idx_hbm, val_hbm, out_hbm,
             iptr_v, idx_v, val_v, rows_v, acc_v, sem):
    _body(x_hbm, iptr_hbm, idx_hbm, val_hbm, out_hbm,
          iptr_v, idx_v, val_v, rows_v, acc_v, sem)


@jax.jit
def kernel(x, indptr, indices, values):
    iptr32 = indptr.astype(jnp.int32)
    iptr_pad = jnp.concatenate(
        [iptr32, jnp.full((IPTR_LEN - (N + 1),), E, jnp.int32)])
    idx_pad = jnp.concatenate(
        [indices.astype(jnp.int32), jnp.zeros((EP - E,), jnp.int32)])
    val_pad = jnp.concatenate(
        [values, jnp.zeros((EP - E,), jnp.float32)])
    out_flat = _sc_spmm(x, iptr_pad, idx_pad, val_pad)
    return out_flat.reshape(NPAD, D)[:N]


# speedup vs baseline: 5.0429x; 5.0429x over previous
import functools

import jax
import jax.numpy as jnp
from jax import lax
from jax.experimental import pallas as pl
from jax.experimental.pallas import tpu as pltpu
from jax.experimental.pallas import tpu_sc as plsc

N = 10000
E = 320000
D = 128

NW = 32
RPW = 320
NPAD = NW * RPW
C = 128
IPTR_BUF = 336
IPTR_LEN = (NW - 1) * RPW + IPTR_BUF
EP = E + 2 * C


def _body(x_hbm, iptr_hbm, idx_hbm, val_hbm, out_hbm,
          iptr_v, idx_v, val_v, rows_v, acc_v, sem):
    cid = lax.axis_index("c")
    sid = lax.axis_index("s")
    w = sid * 2 + cid
    r0 = pl.multiple_of(w * RPW, RPW)

    pltpu.sync_copy(iptr_hbm.at[pl.ds(r0, IPTR_BUF)], iptr_v)

    zeros16 = jnp.zeros((16,), jnp.float32)

    def zero_body(i, carry):
        acc_v[pl.ds(pl.multiple_of(i * 16, 16), 16)] = zeros16
        return carry

    lax.fori_loop(0, RPW * D // 16, zero_body, 0)

    e_lo = iptr_v[pl.ds(0, 16)][0]
    e_hi = iptr_v[pl.ds(RPW, 16)][0]
    a0 = lax.bitwise_and(e_lo, jnp.int32(-16))
    nch = (e_hi - a0 + (C - 1)) // C
    iota16 = lax.iota(jnp.int32, 16)

    def chunk_body(k, carry):
        s = pl.multiple_of(a0 + k * C, 16)
        pltpu.sync_copy(idx_hbm.at[pl.ds(s, C)], idx_v)
        pltpu.sync_copy(val_hbm.at[pl.ds(s, C)], val_v)
        pltpu.async_copy(x_hbm.at[idx_v], rows_v, sem).wait()

        for g in range(C // 16):
            pvec = s + g * 16 + iota16
            vblk = val_v[pl.ds(g * 16, 16)]
            in_range = (pvec >= e_lo) & (pvec < e_hi)
            vblk = jnp.where(in_range, vblk, 0.0)

            lo = jnp.zeros((16,), jnp.int32)
            hi = jnp.full((16,), RPW - 1, jnp.int32)
            for _ in range(9):
                mid = (lo + hi + 1) >> 1
                t = plsc.load_gather(iptr_v, [mid])
                pred = t <= pvec
                lo = jnp.where(pred, mid, lo)
                hi = jnp.where(pred, hi, mid - 1)
            rofs = lo * D

            for l in range(16):
                j = g * 16 + l
                ab = rofs[l]
                vv = jnp.full((16,), vblk[l])
                for d in range(8):
                    xv = rows_v[j, pl.ds(d * 16, 16)]
                    plsc.addupdate(acc_v.at[pl.ds(ab + d * 16, 16)],
                                   xv * vv)
        return carry

    lax.fori_loop(0, nch, chunk_body, 0)

    pltpu.sync_copy(acc_v, out_hbm.at[pl.ds(pl.multiple_of(r0 * D, 16),
                                            RPW * D)])


@functools.partial(
    pl.kernel,
    out_type=jax.ShapeDtypeStruct((NPAD * D,), jnp.float32),
    mesh=plsc.VectorSubcoreMesh(core_axis_name="c", subcore_axis_name="s"),
    scratch_types=[
        pltpu.VMEM((IPTR_BUF,), jnp.int32),
        pltpu.VMEM((C,), jnp.int32),
        pltpu.VMEM((C,), jnp.float32),
        pltpu.VMEM((C, D), jnp.float32),
        pltpu.VMEM((RPW * D,), jnp.float32),
        pltpu.SemaphoreType.DMA,
    ],
    compiler_params=pltpu.CompilerParams(needs_layout_passes=False),
)
def _sc_spmm(x_hbm, iptr_hbm, idx_hbm, val_hbm, out_hbm,
             iptr_v, idx_v, val_v, rows_v, acc_v, sem):
    _body(x_hbm, iptr_hbm, idx_hbm, val_hbm, out_hbm,
          iptr_v, idx_v, val_v, rows_v, acc_v, sem)


@jax.jit
def kernel(x, indptr, indices, values):
    iptr32 = indptr.astype(jnp.int32)
    iptr_pad = jnp.concatenate(
        [iptr32, jnp.full((IPTR_LEN - (N + 1),), E, jnp.int32)])
    idx_pad = jnp.concatenate(
        [indices.astype(jnp.int32), jnp.zeros((EP - E,), jnp.int32)])
    val_pad = jnp.concatenate(
        [values, jnp.zeros((EP - E,), jnp.float32)])
    out_flat = _sc_spmm(x, iptr_pad, idx_pad, val_pad)
    return out_flat.reshape(NPAD, D)[:N]
